# Initial kernel scaffold; baseline (speedup 1.0000x reference)
import jax
import jax.numpy as jnp
from jax.experimental import pallas as pl


def kernel(x, edge_index, W0, as0, ad0, b0, W1, as1, ad1, b1, W2, as2, ad2, b2, Wp, bp):
    raise NotImplementedError("write your pallas kernel here")


# speedup vs baseline: 1.1317x; 1.1317x over previous
import functools

import jax
import jax.numpy as jnp
from jax.experimental import pallas as pl
from jax.experimental.pallas import tpu as pltpu

NEG_SLOPE = 0.2
N_NODES = 10000
ROW_BLK = 1000


def _mm_body(x_ref, w_ref, o_ref):
    o_ref[...] = jnp.dot(x_ref[...], w_ref[...],
                         preferred_element_type=jnp.float32)


def _mm(x, w):
    n, k = x.shape
    m = w.shape[1]
    grid = (n // ROW_BLK,)
    return pl.pallas_call(
        _mm_body,
        grid=grid,
        in_specs=[
            pl.BlockSpec((ROW_BLK, k), lambda i: (i, 0)),
            pl.BlockSpec((k, m), lambda i: (0, 0)),
        ],
        out_specs=pl.BlockSpec((ROW_BLK, m), lambda i: (i, 0)),
        out_shape=jax.ShapeDtypeStruct((n, m), jnp.float32),
    )(x, w)


def _gat_layer(x, src, dst, W, a_src, a_dst, b, H, C):
    n = x.shape[0]
    h2 = _mm(x, W)
    h = h2.reshape(n, H, C)
    alpha_src = (h * a_src[None]).sum(-1)
    alpha_dst = (h * a_dst[None]).sum(-1)
    alpha = alpha_src[src] + alpha_dst[dst]
    alpha = jax.nn.leaky_relu(alpha, NEG_SLOPE)
    e = jnp.exp(alpha)
    denom = jax.ops.segment_sum(e, dst, num_segments=n)
    numer = jax.ops.segment_sum(h[src] * e[..., None], dst, num_segments=n)
    out = numer / (denom[..., None] + 1e-16)
    return out.reshape(n, H * C) + b


def kernel(x, edge_index, W0, as0, ad0, b0, W1, as1, ad1, b1,
           W2, as2, ad2, b2, Wp, bp):
    n = x.shape[0]
    loop = jnp.arange(n, dtype=edge_index.dtype)
    src = jnp.concatenate([edge_index[0], loop])
    dst = jnp.concatenate([edge_index[1], loop])
    h = _gat_layer(x, src, dst, W0, as0, ad0, b0, 8, 16)
    h = jax.nn.elu(h)
    h = _gat_layer(h, src, dst, W1, as1, ad1, b1, 8, 16)
    h = jax.nn.elu(h)
    h = _gat_layer(h, src, dst, W2, as2, ad2, b2, 1, 128)
    h = _mm(h, Wp) + bp
    nrm = jnp.sqrt((h * h).sum(-1, keepdims=True))
    return h / jnp.maximum(nrm, 1e-12)


# speedup vs baseline: 13.1368x; 11.6078x over previous
import functools

import jax
import jax.numpy as jnp
from jax import lax
from jax.experimental import pallas as pl
from jax.experimental.pallas import tpu as pltpu
from jax.experimental.pallas import tpu_sc as plsc

NEG_SLOPE = 0.2
_N = 10000
_EREAL = 330000
_B = 64
_NCH = 324
_EPT = _B * _NCH
_EP = _EPT * 16
_NP = 10240
_NPT = _NP // 16
_ROW_BLK = 1000


def _mm_body(x_ref, w_ref, o_ref):
    o_ref[...] = jnp.dot(x_ref[...], w_ref[...],
                         preferred_element_type=jnp.float32)


def _mm(x, w):
    n, k = x.shape
    m = w.shape[1]
    return pl.pallas_call(
        _mm_body,
        grid=(n // _ROW_BLK,),
        in_specs=[
            pl.BlockSpec((_ROW_BLK, k), lambda i: (i, 0)),
            pl.BlockSpec((k, m), lambda i: (0, 0)),
        ],
        out_specs=pl.BlockSpec((_ROW_BLK, m), lambda i: (i, 0)),
        out_shape=jax.ShapeDtypeStruct((n, m), jnp.float32),
    )(x, w)


def _fused_body(n0_ref, n1_ref, dx_ref, b_ref, w_ref, o_ref):
    h = jnp.concatenate([n0_ref[...], n1_ref[...]], axis=1)
    h = h / (dx_ref[...] + 1e-16) + b_ref[...]
    h = jnp.where(h > 0, h, jnp.exp(h) - 1.0)
    o_ref[...] = jnp.dot(h, w_ref[...], preferred_element_type=jnp.float32)


def _mm_fused(n0, n1, dexp, b, w):
    n = n0.shape[0]
    m = w.shape[1]
    return pl.pallas_call(
        _fused_body,
        grid=(n // _ROW_BLK,),
        in_specs=[
            pl.BlockSpec((_ROW_BLK, 64), lambda i: (i, 0)),
            pl.BlockSpec((_ROW_BLK, 64), lambda i: (i, 0)),
            pl.BlockSpec((_ROW_BLK, 128), lambda i: (i, 0)),
            pl.BlockSpec((1, 128), lambda i: (0, 0)),
            pl.BlockSpec((128, m), lambda i: (0, 0)),
        ],
        out_specs=pl.BlockSpec((_ROW_BLK, m), lambda i: (i, 0)),
        out_shape=jax.ShapeDtypeStruct((n, m), jnp.float32),
    )(n0, n1, dexp, b, w)


def _final_body(n0_ref, n1_ref, dx_ref, b_ref, wp_ref, bp_ref, o_ref):
    h = jnp.concatenate([n0_ref[...], n1_ref[...]], axis=1)
    h = h / (dx_ref[...] + 1e-16) + b_ref[...]
    o = jnp.dot(h, wp_ref[...], preferred_element_type=jnp.float32)
    o = o + bp_ref[...]
    nrm = jnp.sqrt(jnp.sum(o * o, axis=1, keepdims=True))
    o_ref[...] = o / jnp.maximum(nrm, 1e-12)


def _mm_final(n0, n1, dexp, b, wp, bp):
    n = n0.shape[0]
    return pl.pallas_call(
        _final_body,
        grid=(n // _ROW_BLK,),
        in_specs=[
            pl.BlockSpec((_ROW_BLK, 64), lambda i: (i, 0)),
            pl.BlockSpec((_ROW_BLK, 64), lambda i: (i, 0)),
            pl.BlockSpec((_ROW_BLK, 128), lambda i: (i, 0)),
            pl.BlockSpec((1, 128), lambda i: (0, 0)),
            pl.BlockSpec((128, 128), lambda i: (0, 0)),
            pl.BlockSpec((1, 128), lambda i: (0, 0)),
        ],
        out_specs=pl.BlockSpec((_ROW_BLK, 128), lambda i: (i, 0)),
        out_shape=jax.ShapeDtypeStruct((n, 128), jnp.float32),
    )(n0, n1, dexp, b, wp, bp)


def _make_edge_kernel():
    mesh = plsc.VectorSubcoreMesh(core_axis_name="c", subcore_axis_name="s")

    @functools.partial(
        pl.kernel,
        mesh=mesh,
        compiler_params=pltpu.CompilerParams(
            needs_layout_passes=False, use_tc_tiling_on_sc=False),
        out_type=(
            jax.ShapeDtypeStruct((2 * _NP, 64), jnp.float32),
            jax.ShapeDtypeStruct((2 * _NP, 8), jnp.float32),
        ),
        scratch_types=[
            pltpu.VMEM((_N * 8,), jnp.float32),
            pltpu.VMEM((_B,), jnp.int32),
            pltpu.VMEM((_B,), jnp.int32),
            pltpu.VMEM((_B,), jnp.int32),
            pltpu.VMEM((_B, 64), jnp.float32),
            pltpu.VMEM((_B, 8), jnp.float32),
            pltpu.VMEM_SHARED((_NP, 64), jnp.float32),
            pltpu.VMEM_SHARED((_NP, 8), jnp.float32),
            pltpu.SemaphoreType.DMA,
        ],
    )
    def edge_kernel(h2, att2, srcp, dstp, z64, z4, numer_o, denom_o,
                    att_v, src_v, dst_v, srca_v, rows_v, er_v,
                    num_s, den_s, sem):
        c = lax.axis_index("c")
        s = lax.axis_index("s")
        row0 = s * _NPT
        pltpu.sync_copy(z64.at[pl.ds(row0, _NPT)], num_s.at[pl.ds(row0, _NPT)])
        pltpu.sync_copy(z4.at[pl.ds(row0, _NPT)], den_s.at[pl.ds(row0, _NPT)])
        pltpu.sync_copy(att2.at[pl.ds(c * _N * 8, _N * 8)], att_v)
        plsc.subcore_barrier()

        ebase = s * _EPT
        coff = c * _N
        iota = lax.iota(jnp.int32, 16)

        def chunk(ki, carry):
            cb = ebase + ki * _B
            pltpu.sync_copy(srcp.at[pl.ds(cb, _B)], src_v)
            pltpu.sync_copy(dstp.at[pl.ds(cb, _B)], dst_v)

            def grp_a(g, _):
                off = g * 16
                s16 = src_v[pl.ds(off, 16)]
                d16 = dst_v[pl.ds(off, 16)]
                srca_v[pl.ds(off, 16)] = s16 + coff
                valid = (cb + off + iota) < _EREAL
                gi = off + iota
                s8 = s16 * 8
                d8 = d16 * 8
                for h in range(4):
                    hc = jnp.full((16,), h, jnp.int32)
                    asg = plsc.load_gather(att_v, [s8 + h])
                    adg = plsc.load_gather(att_v, [d8 + (h + 4)])
                    a = asg + adg
                    a = jnp.where(a >= 0, a, a * NEG_SLOPE)
                    e = jnp.where(valid, jnp.exp(a), 0.0)
                    plsc.store_scatter(er_v, [gi, hc], e)
                    plsc.store_scatter(er_v, [gi, hc + 4], e)
                return 0

            lax.fori_loop(0, _B // 16, grp_a, 0)
            pltpu.async_copy(h2.at[srca_v], rows_v, sem).wait()

            def grp_b(g, _):
                gi = g * 16 + iota
                for hh in range(4):
                    w = plsc.load_gather(er_v, [gi, jnp.full((16,), hh, jnp.int32)])
                    for cc in range(16):
                        col = jnp.full((16,), hh * 16 + cc, jnp.int32)
                        v = plsc.load_gather(rows_v, [gi, col])
                        plsc.store_scatter(rows_v, [gi, col], v * w)
                return 0

            lax.fori_loop(0, _B // 16, grp_b, 0)
            pltpu.sync_copy(rows_v, num_s.at[dst_v], add=True)
            pltpu.sync_copy(er_v, den_s.at[dst_v], add=True)
            return 0

        lax.fori_loop(0, _NCH, chunk, 0)
        plsc.subcore_barrier()
        pltpu.sync_copy(num_s.at[pl.ds(row0, _NPT)],
                        numer_o.at[pl.ds(c * _NP + row0, _NPT)])
        pltpu.sync_copy(den_s.at[pl.ds(row0, _NPT)],
                        denom_o.at[pl.ds(c * _NP + row0, _NPT)])

    return edge_kernel


_edge = _make_edge_kernel()


def _blockdiag(a, H, C):
    r = jnp.arange(H * C)
    return jnp.zeros((H * C, H), a.dtype).at[r, r // C].set(a.reshape(-1))


def _edge_pass(hfull, srcp, dstp, z64, z4):
    h = hfull[:, :128]
    asn = hfull[:, 128:136]
    adn = hfull[:, 136:144]
    h2 = jnp.concatenate([h[:, :64], h[:, 64:]], axis=0)
    att2 = jnp.concatenate([
        jnp.concatenate([asn[:, :4], adn[:, :4]], axis=1),
        jnp.concatenate([asn[:, 4:], adn[:, 4:]], axis=1),
    ], axis=0)
    numer, denom = _edge(h2, att2.reshape(-1), srcp, dstp, z64, z4)
    n0 = numer[:_N]
    n1 = numer[_NP:_NP + _N]
    dcat = jnp.concatenate([denom[:_N, :4], denom[_NP:_NP + _N, :4]], axis=1)
    dexp = jnp.repeat(dcat, 16, axis=1)
    return n0, n1, dexp


def kernel(x, edge_index, W0, as0, ad0, b0, W1, as1, ad1, b1,
           W2, as2, ad2, b2, Wp, bp):
    loop = jnp.arange(_N, dtype=edge_index.dtype)
    pad = jnp.zeros((_EP - _EREAL,), edge_index.dtype)
    srcp = jnp.concatenate([edge_index[0], loop, pad]).astype(jnp.int32)
    dstp = jnp.concatenate([edge_index[1], loop, pad]).astype(jnp.int32)
    z64 = jnp.zeros((_NP, 64), jnp.float32)
    z4 = jnp.zeros((_NP, 8), jnp.float32)

    wf0 = jnp.concatenate(
        [W0, W0 @ _blockdiag(as0, 8, 16), W0 @ _blockdiag(ad0, 8, 16)], axis=1)
    wf1 = jnp.concatenate(
        [W1, W1 @ _blockdiag(as1, 8, 16), W1 @ _blockdiag(ad1, 8, 16)], axis=1)
    a2s = jnp.tile(as2.reshape(128, 1), (1, 8))
    a2d = jnp.tile(ad2.reshape(128, 1), (1, 8))
    wf2 = jnp.concatenate([W2, W2 @ a2s, W2 @ a2d], axis=1)

    hfull = _mm(x, wf0)
    n0, n1, dexp = _edge_pass(hfull, srcp, dstp, z64, z4)
    hfull = _mm_fused(n0, n1, dexp, b0.reshape(1, 128), wf1)
    n0, n1, dexp = _edge_pass(hfull, srcp, dstp, z64, z4)
    hfull = _mm_fused(n0, n1, dexp, b1.reshape(1, 128), wf2)
    n0, n1, dexp = _edge_pass(hfull, srcp, dstp, z64, z4)
    return _mm_final(n0, n1, dexp, b2.reshape(1, 128),
                     Wp, bp.reshape(1, 128))


# speedup vs baseline: 21.1103x; 1.6070x over previous
import functools

import jax
import jax.numpy as jnp
from jax import lax
from jax.experimental import pallas as pl
from jax.experimental.pallas import tpu as pltpu
from jax.experimental.pallas import tpu_sc as plsc

NEG_SLOPE = 0.2
_N = 10000
_EREAL = 330000
_B = 128
_NCH = 162
_EPT = _B * _NCH
_EP = _EPT * 16
_NP = 10016
_NPT = _NP // 16
_ROW_BLK = 1000


def _mm_body(x_ref, w_ref, o_ref):
    o_ref[...] = jnp.dot(x_ref[...], w_ref[...],
                         preferred_element_type=jnp.float32)


def _mm(x, w):
    n, k = x.shape
    m = w.shape[1]
    return pl.pallas_call(
        _mm_body,
        grid=(n // _ROW_BLK,),
        in_specs=[
            pl.BlockSpec((_ROW_BLK, k), lambda i: (i, 0)),
            pl.BlockSpec((k, m), lambda i: (0, 0)),
        ],
        out_specs=pl.BlockSpec((_ROW_BLK, m), lambda i: (i, 0)),
        out_shape=jax.ShapeDtypeStruct((n, m), jnp.float32),
    )(x, w)


def _fused_body(n0_ref, n1_ref, dx_ref, b_ref, w_ref, o_ref):
    h = jnp.concatenate([n0_ref[...], n1_ref[...]], axis=1)
    h = h / (dx_ref[...] + 1e-16) + b_ref[...]
    h = jnp.where(h > 0, h, jnp.exp(h) - 1.0)
    o_ref[...] = jnp.dot(h, w_ref[...], preferred_element_type=jnp.float32)


def _mm_fused(n0, n1, dexp, b, w):
    n = n0.shape[0]
    m = w.shape[1]
    return pl.pallas_call(
        _fused_body,
        grid=(n // _ROW_BLK,),
        in_specs=[
            pl.BlockSpec((_ROW_BLK, 64), lambda i: (i, 0)),
            pl.BlockSpec((_ROW_BLK, 64), lambda i: (i, 0)),
            pl.BlockSpec((_ROW_BLK, 128), lambda i: (i, 0)),
            pl.BlockSpec((1, 128), lambda i: (0, 0)),
            pl.BlockSpec((128, m), lambda i: (0, 0)),
        ],
        out_specs=pl.BlockSpec((_ROW_BLK, m), lambda i: (i, 0)),
        out_shape=jax.ShapeDtypeStruct((n, m), jnp.float32),
    )(n0, n1, dexp, b, w)


def _final_body(n0_ref, n1_ref, dx_ref, b_ref, wp_ref, bp_ref, o_ref):
    h = jnp.concatenate([n0_ref[...], n1_ref[...]], axis=1)
    h = h / (dx_ref[...] + 1e-16) + b_ref[...]
    o = jnp.dot(h, wp_ref[...], preferred_element_type=jnp.float32)
    o = o + bp_ref[...]
    nrm = jnp.sqrt(jnp.sum(o * o, axis=1, keepdims=True))
    o_ref[...] = o / jnp.maximum(nrm, 1e-12)


def _mm_final(n0, n1, dexp, b, wp, bp):
    n = n0.shape[0]
    return pl.pallas_call(
        _final_body,
        grid=(n // _ROW_BLK,),
        in_specs=[
            pl.BlockSpec((_ROW_BLK, 64), lambda i: (i, 0)),
            pl.BlockSpec((_ROW_BLK, 64), lambda i: (i, 0)),
            pl.BlockSpec((_ROW_BLK, 128), lambda i: (i, 0)),
            pl.BlockSpec((1, 128), lambda i: (0, 0)),
            pl.BlockSpec((128, 128), lambda i: (0, 0)),
            pl.BlockSpec((1, 128), lambda i: (0, 0)),
        ],
        out_specs=pl.BlockSpec((_ROW_BLK, 128), lambda i: (i, 0)),
        out_shape=jax.ShapeDtypeStruct((n, 128), jnp.float32),
    )(n0, n1, dexp, b, wp, bp)


def _make_edge_kernel():
    mesh = plsc.VectorSubcoreMesh(core_axis_name="c", subcore_axis_name="s")

    @functools.partial(
        pl.kernel,
        mesh=mesh,
        compiler_params=pltpu.CompilerParams(
            needs_layout_passes=False, use_tc_tiling_on_sc=False),
        out_type=jax.ShapeDtypeStruct((2 * _NP, 72), jnp.float32),
        scratch_types=[
            pltpu.VMEM((_B,), jnp.int32),
            pltpu.VMEM((_B,), jnp.int32),
            pltpu.VMEM((_B,), jnp.int32),
            pltpu.VMEM((_B,), jnp.int32),
            pltpu.VMEM((_B, 64), jnp.float32),
            pltpu.VMEM((_B, 64), jnp.float32),
            pltpu.VMEM((_B, 8), jnp.float32),
            pltpu.VMEM((_B, 8), jnp.float32),
            pltpu.VMEM((_B, 8), jnp.float32),
            pltpu.VMEM((_B, 8), jnp.float32),
            pltpu.VMEM((_B, 72), jnp.float32),
            pltpu.VMEM((_B, 72), jnp.float32),
            pltpu.VMEM_SHARED((_N, 64), jnp.float32),
            pltpu.VMEM_SHARED((_N, 8), jnp.float32),
            pltpu.VMEM_SHARED((_NP, 72), jnp.float32),
            pltpu.SemaphoreType.DMA,
            pltpu.SemaphoreType.DMA,
            pltpu.SemaphoreType.DMA,
            pltpu.SemaphoreType.DMA,
        ],
    )
    def edge_kernel(h2, att2, srcp, dstp, z72, acc_o,
                    src0, src1, dst0, dst1, rows0, rows1,
                    asr0, asr1, adr0, adr1, comb0, comb1,
                    hs, att_s, acc_s, gs0, gs1, ns0, ns1):
        c = lax.axis_index("c")
        s = lax.axis_index("s")
        row0 = s * _NPT
        pltpu.sync_copy(z72.at[pl.ds(row0, _NPT)], acc_s.at[pl.ds(row0, _NPT)])
        pltpu.sync_copy(h2.at[pl.ds(c * _N + s * 625, 625)],
                        hs.at[pl.ds(s * 625, 625)])
        pltpu.sync_copy(att2.at[pl.ds(c * _N + s * 625, 625)],
                        att_s.at[pl.ds(s * 625, 625)])
        plsc.subcore_barrier()

        ebase = s * _EPT
        iota = lax.iota(jnp.int32, 16)
        bufs = ((src0, dst0, rows0, asr0, adr0, comb0, gs0, ns0),
                (src1, dst1, rows1, asr1, adr1, comb1, gs1, ns1))

        def fire(k, src_b, dst_b, rows_b, asr_b, adr_b, gsem):
            cb = ebase + k * _B
            pltpu.sync_copy(srcp.at[pl.ds(cb, _B)], src_b)
            pltpu.sync_copy(dstp.at[pl.ds(cb, _B)], dst_b)
            pltpu.async_copy(hs.at[src_b], rows_b, gsem)
            pltpu.async_copy(att_s.at[src_b], asr_b, gsem)
            pltpu.async_copy(att_s.at[dst_b], adr_b, gsem)

        def finish(k, src_b, dst_b, rows_b, asr_b, adr_b, comb_b, gsem, nsem):
            pltpu.make_async_copy(hs.at[src_b], rows_b, gsem).wait()
            pltpu.make_async_copy(att_s.at[src_b], asr_b, gsem).wait()
            pltpu.make_async_copy(att_s.at[dst_b], adr_b, gsem).wait()
            cb = ebase + k * _B

            def grp(g, _):
                off = g * 16
                gi = off + iota
                valid = (cb + off + iota) < _EREAL
                for hh in range(4):
                    hc = jnp.full((16,), hh, jnp.int32)
                    asg = plsc.load_gather(asr_b, [gi, hc])
                    adg = plsc.load_gather(adr_b, [gi, hc + 4])
                    a = asg + adg
                    a = jnp.where(a >= 0, a, a * NEG_SLOPE)
                    w = jnp.where(valid, jnp.exp(a), 0.0)
                    plsc.store_scatter(comb_b, [gi, hc + 64], w)
                    plsc.store_scatter(comb_b, [gi, hc + 68], w)
                    for cc in range(16):
                        col = jnp.full((16,), hh * 16 + cc, jnp.int32)
                        v = plsc.load_gather(rows_b, [gi, col])
                        plsc.store_scatter(comb_b, [gi, col], v * w)
                return 0

            lax.fori_loop(0, _B // 16, grp, 0)
            pltpu.async_copy(comb_b, acc_s.at[dst_b], nsem, add=True)

        def drain(comb_b, nsem):
            pltpu.make_async_copy(z72.at[pl.ds(0, _B)], comb_b, nsem).wait()

        def pair(k2, carry):
            for prt in (0, 1):
                k = 2 * k2 + prt
                src_b, dst_b, rows_b, asr_b, adr_b, comb_b, gsem, nsem = bufs[prt]
                osrc, odst, orows, oasr, oadr, ocomb, ogsem, onsem = bufs[1 - prt]

                @pl.when(k2 >= 1)
                def _():
                    drain(comb_b, nsem)

                fire(k, src_b, dst_b, rows_b, asr_b, adr_b, gsem)

                if prt == 0:
                    @pl.when(k2 >= 1)
                    def _():
                        finish(k - 1, osrc, odst, orows, oasr, oadr, ocomb,
                               ogsem, onsem)
                else:
                    finish(k - 1, osrc, odst, orows, oasr, oadr, ocomb,
                           ogsem, onsem)
            return 0

        lax.fori_loop(0, _NCH // 2, pair, 0)
        finish(_NCH - 1, src1, dst1, rows1, asr1, adr1, comb1, gs1, ns1)
        drain(comb0, ns0)
        drain(comb1, ns1)
        plsc.subcore_barrier()
        pltpu.sync_copy(acc_s.at[pl.ds(row0, _NPT)],
                        acc_o.at[pl.ds(c * _NP + row0, _NPT)])

    return edge_kernel


_edge = _make_edge_kernel()


def _blockdiag(a, H, C):
    r = jnp.arange(H * C)
    return jnp.zeros((H * C, H), a.dtype).at[r, r // C].set(a.reshape(-1))


def _edge_pass(hfull, srcp, dstp, z72):
    h = hfull[:, :128]
    asn = hfull[:, 128:136]
    adn = hfull[:, 136:144]
    h2 = jnp.concatenate([h[:, :64], h[:, 64:]], axis=0)
    att2 = jnp.concatenate([
        jnp.concatenate([asn[:, :4], adn[:, :4]], axis=1),
        jnp.concatenate([asn[:, 4:], adn[:, 4:]], axis=1),
    ], axis=0)
    acc = _edge(h2, att2, srcp, dstp, z72)
    out0, out1 = acc[:_N], acc[_NP:_NP + _N]
    n0 = out0[:, :64]
    n1 = out1[:, :64]
    dcat = jnp.concatenate([out0[:, 64:68], out1[:, 64:68]], axis=1)
    dexp = jnp.repeat(dcat, 16, axis=1)
    return n0, n1, dexp


def kernel(x, edge_index, W0, as0, ad0, b0, W1, as1, ad1, b1,
           W2, as2, ad2, b2, Wp, bp):
    loop = jnp.arange(_N, dtype=edge_index.dtype)
    pad = jnp.zeros((_EP - _EREAL,), edge_index.dtype)
    srcp = jnp.concatenate([edge_index[0], loop, pad]).astype(jnp.int32)
    dstp = jnp.concatenate([edge_index[1], loop, pad]).astype(jnp.int32)
    z72 = jnp.zeros((_NP, 72), jnp.float32)

    wf0 = jnp.concatenate(
        [W0, W0 @ _blockdiag(as0, 8, 16), W0 @ _blockdiag(ad0, 8, 16)], axis=1)
    wf1 = jnp.concatenate(
        [W1, W1 @ _blockdiag(as1, 8, 16), W1 @ _blockdiag(ad1, 8, 16)], axis=1)
    a2s = jnp.tile(as2.reshape(128, 1), (1, 8))
    a2d = jnp.tile(ad2.reshape(128, 1), (1, 8))
    wf2 = jnp.concatenate([W2, W2 @ a2s, W2 @ a2d], axis=1)

    hfull = _mm(x, wf0)
    n0, n1, dexp = _edge_pass(hfull, srcp, dstp, z72)
    hfull = _mm_fused(n0, n1, dexp, b0.reshape(1, 128), wf1)
    n0, n1, dexp = _edge_pass(hfull, srcp, dstp, z72)
    hfull = _mm_fused(n0, n1, dexp, b1.reshape(1, 128), wf2)
    n0, n1, dexp = _edge_pass(hfull, srcp, dstp, z72)
    return _mm_final(n0, n1, dexp, b2.reshape(1, 128),
                     Wp, bp.reshape(1, 128))


# speedup vs baseline: 38.8896x; 1.8422x over previous
import functools

import jax
import jax.numpy as jnp
from jax import lax
from jax.experimental import pallas as pl
from jax.experimental.pallas import tpu as pltpu
from jax.experimental.pallas import tpu_sc as plsc

NEG_SLOPE = 0.2
_N = 10000
_EREAL = 330000
_B = 128
_NCH = 162
_EPT = _B * _NCH
_EP = _EPT * 16
_NP = 10016
_NPT = _NP // 16
_ROW_BLK = 1000


def _mm_body(x_ref, w_ref, o_ref):
    o_ref[...] = jnp.dot(x_ref[...], w_ref[...],
                         preferred_element_type=jnp.float32)


def _mm(x, w):
    n, k = x.shape
    m = w.shape[1]
    return pl.pallas_call(
        _mm_body,
        grid=(n // _ROW_BLK,),
        in_specs=[
            pl.BlockSpec((_ROW_BLK, k), lambda i: (i, 0)),
            pl.BlockSpec((k, m), lambda i: (0, 0)),
        ],
        out_specs=pl.BlockSpec((_ROW_BLK, m), lambda i: (i, 0)),
        out_shape=jax.ShapeDtypeStruct((n, m), jnp.float32),
    )(x, w)


def _fused_body(n0_ref, n1_ref, dx_ref, b_ref, w_ref, o_ref):
    h = jnp.concatenate([n0_ref[...], n1_ref[...]], axis=1)
    h = h / (dx_ref[...] + 1e-16) + b_ref[...]
    h = jnp.where(h > 0, h, jnp.exp(h) - 1.0)
    o_ref[...] = jnp.dot(h, w_ref[...], preferred_element_type=jnp.float32)


def _mm_fused(n0, n1, dexp, b, w):
    n = n0.shape[0]
    m = w.shape[1]
    return pl.pallas_call(
        _fused_body,
        grid=(n // _ROW_BLK,),
        in_specs=[
            pl.BlockSpec((_ROW_BLK, 64), lambda i: (i, 0)),
            pl.BlockSpec((_ROW_BLK, 64), lambda i: (i, 0)),
            pl.BlockSpec((_ROW_BLK, 128), lambda i: (i, 0)),
            pl.BlockSpec((1, 128), lambda i: (0, 0)),
            pl.BlockSpec((128, m), lambda i: (0, 0)),
        ],
        out_specs=pl.BlockSpec((_ROW_BLK, m), lambda i: (i, 0)),
        out_shape=jax.ShapeDtypeStruct((n, m), jnp.float32),
    )(n0, n1, dexp, b, w)


def _final_body(n0_ref, n1_ref, dx_ref, b_ref, wp_ref, bp_ref, o_ref):
    h = jnp.concatenate([n0_ref[...], n1_ref[...]], axis=1)
    h = h / (dx_ref[...] + 1e-16) + b_ref[...]
    o = jnp.dot(h, wp_ref[...], preferred_element_type=jnp.float32)
    o = o + bp_ref[...]
    nrm = jnp.sqrt(jnp.sum(o * o, axis=1, keepdims=True))
    o_ref[...] = o / jnp.maximum(nrm, 1e-12)


def _mm_final(n0, n1, dexp, b, wp, bp):
    n = n0.shape[0]
    return pl.pallas_call(
        _final_body,
        grid=(n // _ROW_BLK,),
        in_specs=[
            pl.BlockSpec((_ROW_BLK, 64), lambda i: (i, 0)),
            pl.BlockSpec((_ROW_BLK, 64), lambda i: (i, 0)),
            pl.BlockSpec((_ROW_BLK, 128), lambda i: (i, 0)),
            pl.BlockSpec((1, 128), lambda i: (0, 0)),
            pl.BlockSpec((128, 128), lambda i: (0, 0)),
            pl.BlockSpec((1, 128), lambda i: (0, 0)),
        ],
        out_specs=pl.BlockSpec((_ROW_BLK, 128), lambda i: (i, 0)),
        out_shape=jax.ShapeDtypeStruct((n, 128), jnp.float32),
    )(n0, n1, dexp, b, wp, bp)


def _make_edge_kernel():
    mesh = plsc.VectorSubcoreMesh(core_axis_name="c", subcore_axis_name="s")

    @functools.partial(
        pl.kernel,
        mesh=mesh,
        compiler_params=pltpu.CompilerParams(
            needs_layout_passes=False, use_tc_tiling_on_sc=False),
        out_type=jax.ShapeDtypeStruct((2 * _NP, 72), jnp.float32),
        scratch_types=[
            pltpu.VMEM((_B,), jnp.int32),
            pltpu.VMEM((_B,), jnp.int32),
            pltpu.VMEM((_B,), jnp.int32),
            pltpu.VMEM((_B,), jnp.int32),
            pltpu.VMEM((_B, 64), jnp.float32),
            pltpu.VMEM((_B, 64), jnp.float32),
            pltpu.VMEM((_B, 8), jnp.float32),
            pltpu.VMEM((_B, 8), jnp.float32),
            pltpu.VMEM((_B, 8), jnp.float32),
            pltpu.VMEM((_B, 8), jnp.float32),
            pltpu.VMEM((_B, 72), jnp.float32),
            pltpu.VMEM((_B, 72), jnp.float32),
            pltpu.VMEM_SHARED((_N, 64), jnp.float32),
            pltpu.VMEM_SHARED((_N, 8), jnp.float32),
            pltpu.VMEM_SHARED((_NP, 72), jnp.float32),
            pltpu.SemaphoreType.DMA,
            pltpu.SemaphoreType.DMA,
            pltpu.SemaphoreType.DMA,
            pltpu.SemaphoreType.DMA,
        ],
    )
    def edge_kernel(h2, att2, srcp, dstp, z72, acc_o,
                    src0, src1, dst0, dst1, rows0, rows1,
                    asr0, asr1, adr0, adr1, comb0, comb1,
                    hs, att_s, acc_s, gs0, gs1, ns0, ns1):
        c = lax.axis_index("c")
        s = lax.axis_index("s")
        row0 = s * _NPT
        pltpu.sync_copy(z72.at[pl.ds(row0, _NPT)], acc_s.at[pl.ds(row0, _NPT)])
        pltpu.sync_copy(h2.at[pl.ds(c * _N + s * 625, 625)],
                        hs.at[pl.ds(s * 625, 625)])
        pltpu.sync_copy(att2.at[pl.ds(c * _N + s * 625, 625)],
                        att_s.at[pl.ds(s * 625, 625)])
        plsc.subcore_barrier()

        ebase = s * _EPT
        iota = lax.iota(jnp.int32, 16)
        bufs = ((src0, dst0, rows0, asr0, adr0, comb0, gs0, ns0),
                (src1, dst1, rows1, asr1, adr1, comb1, gs1, ns1))

        def fire(k, src_b, dst_b, rows_b, asr_b, adr_b, gsem):
            cb = ebase + k * _B
            pltpu.sync_copy(srcp.at[pl.ds(cb, _B)], src_b)
            pltpu.sync_copy(dstp.at[pl.ds(cb, _B)], dst_b)
            pltpu.async_copy(hs.at[src_b], rows_b, gsem)
            pltpu.async_copy(att_s.at[src_b], asr_b, gsem)
            pltpu.async_copy(att_s.at[dst_b], adr_b, gsem)

        def finish(k, src_b, dst_b, rows_b, asr_b, adr_b, comb_b,
                   gsem, nsem):
            pltpu.make_async_copy(hs.at[src_b], rows_b, gsem).wait()
            pltpu.make_async_copy(att_s.at[src_b], asr_b, gsem).wait()
            pltpu.make_async_copy(att_s.at[dst_b], adr_b, gsem).wait()
            cb = ebase + k * _B

            def grp(g, _):
                off = g * 16
                gi = off + iota
                valid = (cb + off + iota) < _EREAL
                for hh in range(4):
                    hc = jnp.full((16,), hh, jnp.int32)
                    asg = plsc.load_gather(asr_b, [gi, hc])
                    adg = plsc.load_gather(adr_b, [gi, hc + 4])
                    a = asg + adg
                    a = jnp.where(a >= 0, a, a * NEG_SLOPE)
                    w = jnp.where(valid, jnp.exp(a), 0.0)
                    plsc.store_scatter(comb_b, [gi, hc + 64], w)
                    plsc.store_scatter(comb_b, [gi, hc + 68], w)
                    for i in range(16):
                        b = off + i
                        w16 = jnp.take_along_axis(
                            w, jnp.full((16,), i, jnp.int32), axis=0)
                        v = rows_b[b, pl.ds(hh * 16, 16)]
                        comb_b[b, pl.ds(hh * 16, 16)] = v * w16
                return 0

            lax.fori_loop(0, _B // 16, grp, 0)
            pltpu.async_copy(comb_b, acc_s.at[dst_b], nsem, add=True)

        def drain(comb_b, nsem):
            pltpu.make_async_copy(z72.at[pl.ds(0, _B)], comb_b, nsem).wait()

        def pair(k2, carry):
            for prt in (0, 1):
                k = 2 * k2 + prt
                (src_b, dst_b, rows_b, asr_b, adr_b, comb_b,
                 gsem, nsem) = bufs[prt]
                (osrc, odst, orows, oasr, oadr, ocomb,
                 ogsem, onsem) = bufs[1 - prt]

                @pl.when(k2 >= 1)
                def _():
                    drain(comb_b, nsem)

                fire(k, src_b, dst_b, rows_b, asr_b, adr_b, gsem)

                if prt == 0:
                    @pl.when(k2 >= 1)
                    def _():
                        finish(k - 1, osrc, odst, orows, oasr, oadr, ocomb,
                               ogsem, onsem)
                else:
                    finish(k - 1, osrc, odst, orows, oasr, oadr, ocomb,
                           ogsem, onsem)
            return 0

        lax.fori_loop(0, _NCH // 2, pair, 0)
        finish(_NCH - 1, src1, dst1, rows1, asr1, adr1, comb1, gs1, ns1)
        drain(comb0, ns0)
        drain(comb1, ns1)
        plsc.subcore_barrier()
        pltpu.sync_copy(acc_s.at[pl.ds(row0, _NPT)],
                        acc_o.at[pl.ds(c * _NP + row0, _NPT)])

    return edge_kernel


_edge = _make_edge_kernel()


def _blockdiag(a, H, C):
    r = jnp.arange(H * C)
    return jnp.zeros((H * C, H), a.dtype).at[r, r // C].set(a.reshape(-1))


def _edge_pass(hfull, srcp, dstp, z72):
    h = hfull[:, :128]
    asn = hfull[:, 128:136]
    adn = hfull[:, 136:144]
    h2 = jnp.concatenate([h[:, :64], h[:, 64:]], axis=0)
    att2 = jnp.concatenate([
        jnp.concatenate([asn[:, :4], adn[:, :4]], axis=1),
        jnp.concatenate([asn[:, 4:], adn[:, 4:]], axis=1),
    ], axis=0)
    acc = _edge(h2, att2, srcp, dstp, z72)
    out0, out1 = acc[:_N], acc[_NP:_NP + _N]
    n0 = out0[:, :64]
    n1 = out1[:, :64]
    dcat = jnp.concatenate([out0[:, 64:68], out1[:, 64:68]], axis=1)
    dexp = jnp.repeat(dcat, 16, axis=1)
    return n0, n1, dexp


def kernel(x, edge_index, W0, as0, ad0, b0, W1, as1, ad1, b1,
           W2, as2, ad2, b2, Wp, bp):
    loop = jnp.arange(_N, dtype=edge_index.dtype)
    pad = jnp.zeros((_EP - _EREAL,), edge_index.dtype)
    srcp = jnp.concatenate([edge_index[0], loop, pad]).astype(jnp.int32)
    dstp = jnp.concatenate([edge_index[1], loop, pad]).astype(jnp.int32)
    z72 = jnp.zeros((_NP, 72), jnp.float32)

    wf0 = jnp.concatenate(
        [W0, W0 @ _blockdiag(as0, 8, 16), W0 @ _blockdiag(ad0, 8, 16)], axis=1)
    wf1 = jnp.concatenate(
        [W1, W1 @ _blockdiag(as1, 8, 16), W1 @ _blockdiag(ad1, 8, 16)], axis=1)
    a2s = jnp.tile(as2.reshape(128, 1), (1, 8))
    a2d = jnp.tile(ad2.reshape(128, 1), (1, 8))
    wf2 = jnp.concatenate([W2, W2 @ a2s, W2 @ a2d], axis=1)

    hfull = _mm(x, wf0)
    n0, n1, dexp = _edge_pass(hfull, srcp, dstp, z72)
    hfull = _mm_fused(n0, n1, dexp, b0.reshape(1, 128), wf1)
    n0, n1, dexp = _edge_pass(hfull, srcp, dstp, z72)
    hfull = _mm_fused(n0, n1, dexp, b1.reshape(1, 128), wf2)
    n0, n1, dexp = _edge_pass(hfull, srcp, dstp, z72)
    return _mm_final(n0, n1, dexp, b2.reshape(1, 128),
                     Wp, bp.reshape(1, 128))


# speedup vs baseline: 65.8237x; 1.6926x over previous
import functools

import jax
import jax.numpy as jnp
from jax import lax
from jax.experimental import pallas as pl
from jax.experimental.pallas import tpu as pltpu
from jax.experimental.pallas import tpu_sc as plsc

NEG_SLOPE = 0.2
_N = 10000
_EREAL = 330000
_B = 128
_NCH = 162
_EPT = _B * _NCH
_EP = _EPT * 16
_NP = 10016
_NPT = _NP // 16
_ROW_BLK = 1000


def _mm_body(x_ref, w_ref, o_ref):
    o_ref[...] = jnp.dot(x_ref[...], w_ref[...],
                         preferred_element_type=jnp.float32)


def _mm(x, w):
    n, k = x.shape
    m = w.shape[1]
    return pl.pallas_call(
        _mm_body,
        grid=(n // _ROW_BLK,),
        in_specs=[
            pl.BlockSpec((_ROW_BLK, k), lambda i: (i, 0)),
            pl.BlockSpec((k, m), lambda i: (0, 0)),
        ],
        out_specs=pl.BlockSpec((_ROW_BLK, m), lambda i: (i, 0)),
        out_shape=jax.ShapeDtypeStruct((n, m), jnp.float32),
    )(x, w)


def _fused_body(n0_ref, n1_ref, dx_ref, b_ref, w_ref, o_ref):
    h = jnp.concatenate([n0_ref[...], n1_ref[...]], axis=1)
    h = h / (dx_ref[...] + 1e-16) + b_ref[...]
    h = jnp.where(h > 0, h, jnp.exp(h) - 1.0)
    o_ref[...] = jnp.dot(h, w_ref[...], preferred_element_type=jnp.float32)


def _mm_fused(n0, n1, dexp, b, w):
    n = n0.shape[0]
    m = w.shape[1]
    return pl.pallas_call(
        _fused_body,
        grid=(n // _ROW_BLK,),
        in_specs=[
            pl.BlockSpec((_ROW_BLK, 64), lambda i: (i, 0)),
            pl.BlockSpec((_ROW_BLK, 64), lambda i: (i, 0)),
            pl.BlockSpec((_ROW_BLK, 128), lambda i: (i, 0)),
            pl.BlockSpec((1, 128), lambda i: (0, 0)),
            pl.BlockSpec((128, m), lambda i: (0, 0)),
        ],
        out_specs=pl.BlockSpec((_ROW_BLK, m), lambda i: (i, 0)),
        out_shape=jax.ShapeDtypeStruct((n, m), jnp.float32),
    )(n0, n1, dexp, b, w)


def _final_body(n0_ref, n1_ref, dx_ref, b_ref, wp_ref, bp_ref, o_ref):
    h = jnp.concatenate([n0_ref[...], n1_ref[...]], axis=1)
    h = h / (dx_ref[...] + 1e-16) + b_ref[...]
    o = jnp.dot(h, wp_ref[...], preferred_element_type=jnp.float32)
    o = o + bp_ref[...]
    nrm = jnp.sqrt(jnp.sum(o * o, axis=1, keepdims=True))
    o_ref[...] = o / jnp.maximum(nrm, 1e-12)


def _mm_final(n0, n1, dexp, b, wp, bp):
    n = n0.shape[0]
    return pl.pallas_call(
        _final_body,
        grid=(n // _ROW_BLK,),
        in_specs=[
            pl.BlockSpec((_ROW_BLK, 64), lambda i: (i, 0)),
            pl.BlockSpec((_ROW_BLK, 64), lambda i: (i, 0)),
            pl.BlockSpec((_ROW_BLK, 128), lambda i: (i, 0)),
            pl.BlockSpec((1, 128), lambda i: (0, 0)),
            pl.BlockSpec((128, 128), lambda i: (0, 0)),
            pl.BlockSpec((1, 128), lambda i: (0, 0)),
        ],
        out_specs=pl.BlockSpec((_ROW_BLK, 128), lambda i: (i, 0)),
        out_shape=jax.ShapeDtypeStruct((n, 128), jnp.float32),
    )(n0, n1, dexp, b, wp, bp)


def _make_edge_kernel():
    mesh = plsc.VectorSubcoreMesh(core_axis_name="c", subcore_axis_name="s")

    @functools.partial(
        pl.kernel,
        mesh=mesh,
        compiler_params=pltpu.CompilerParams(
            needs_layout_passes=False, use_tc_tiling_on_sc=False),
        out_type=jax.ShapeDtypeStruct((2 * _NP, 72), jnp.float32),
        scratch_types=[
            pltpu.VMEM((2, _B), jnp.int32),
            pltpu.VMEM((2, _B), jnp.int32),
            pltpu.VMEM((_B, 8), jnp.float32),
            pltpu.VMEM((_B, 8), jnp.float32),
            pltpu.VMEM((_B, 72), jnp.float32),
            pltpu.VMEM((_B, 72), jnp.float32),
            pltpu.VMEM_SHARED((_N, 72), jnp.float32),
            pltpu.VMEM_SHARED((_N, 8), jnp.float32),
            pltpu.VMEM_SHARED((_NP, 72), jnp.float32),
            pltpu.SemaphoreType.DMA,
            pltpu.SemaphoreType.DMA,
            pltpu.SemaphoreType.DMA,
            pltpu.SemaphoreType.DMA,
        ],
    )
    def edge_kernel(h2e, att2, sdp, z72, acc_o,
                    sd0, sd1, adr0, adr1, comb0, comb1,
                    hs, att_s, acc_s, gs0, gs1, ns0, ns1):
        c = lax.axis_index("c")
        s = lax.axis_index("s")
        row0 = s * _NPT
        pltpu.sync_copy(z72.at[pl.ds(row0, _NPT)], acc_s.at[pl.ds(row0, _NPT)])
        pltpu.sync_copy(h2e.at[pl.ds(c * _N + s * 625, 625)],
                        hs.at[pl.ds(s * 625, 625)])
        pltpu.sync_copy(att2.at[pl.ds(c * _N + s * 625, 625)],
                        att_s.at[pl.ds(s * 625, 625)])
        plsc.subcore_barrier()

        iota = lax.iota(jnp.int32, 16)
        bufs = ((sd0, adr0, comb0, gs0, ns0),
                (sd1, adr1, comb1, gs1, ns1))

        def fire(k, sd_b, adr_b, comb_b, gsem):
            pltpu.sync_copy(sdp.at[pl.ds((s * _NCH + k) * 2, 2)], sd_b)
            pltpu.async_copy(hs.at[sd_b.at[0]], comb_b, gsem)
            pltpu.async_copy(att_s.at[sd_b.at[1]], adr_b, gsem)

        def finish(k, sd_b, adr_b, comb_b, gsem, nsem):
            pltpu.make_async_copy(hs.at[sd_b.at[0]], comb_b, gsem).wait()
            pltpu.make_async_copy(att_s.at[sd_b.at[1]], adr_b, gsem).wait()
            cb = s * _EPT + k * _B

            def grp(g, _):
                off = g * 16
                gi = off + iota
                valid = (cb + off + iota) < _EREAL
                for hh in range(4):
                    hc = jnp.full((16,), hh, jnp.int32)
                    asg = plsc.load_gather(comb_b, [gi, hc + 64])
                    adg = plsc.load_gather(adr_b, [gi, hc + 4])
                    a = asg + adg
                    a = jnp.where(a >= 0, a, a * NEG_SLOPE)
                    w = jnp.where(valid, jnp.exp(a), 0.0)
                    plsc.store_scatter(comb_b, [gi, hc + 64], w)
                    plsc.store_scatter(comb_b, [gi, hc + 68], w)
                    for i in range(16):
                        b = off + i
                        w16 = jnp.take_along_axis(
                            w, jnp.full((16,), i, jnp.int32), axis=0)
                        v = comb_b[b, pl.ds(hh * 16, 16)]
                        comb_b[b, pl.ds(hh * 16, 16)] = v * w16
                return 0

            lax.fori_loop(0, _B // 16, grp, 0)
            pltpu.async_copy(comb_b, acc_s.at[sd_b.at[1]], nsem, add=True)

        def drain(comb_b, nsem):
            pltpu.make_async_copy(z72.at[pl.ds(0, _B)], comb_b, nsem).wait()

        def pair(k2, carry):
            for prt in (0, 1):
                k = 2 * k2 + prt
                sd_b, adr_b, comb_b, gsem, nsem = bufs[prt]
                osd, oadr, ocomb, ogsem, onsem = bufs[1 - prt]

                @pl.when(k2 >= 1)
                def _():
                    drain(comb_b, nsem)

                fire(k, sd_b, adr_b, comb_b, gsem)

                if prt == 0:
                    @pl.when(k2 >= 1)
                    def _():
                        finish(k - 1, osd, oadr, ocomb, ogsem, onsem)
                else:
                    finish(k - 1, osd, oadr, ocomb, ogsem, onsem)
            return 0

        lax.fori_loop(0, _NCH // 2, pair, 0)
        finish(_NCH - 1, sd1, adr1, comb1, gs1, ns1)
        drain(comb0, ns0)
        drain(comb1, ns1)
        plsc.subcore_barrier()
        pltpu.sync_copy(acc_s.at[pl.ds(row0, _NPT)],
                        acc_o.at[pl.ds(c * _NP + row0, _NPT)])

    return edge_kernel


_edge = _make_edge_kernel()


def _blockdiag(a, H, C):
    r = jnp.arange(H * C)
    return jnp.zeros((H * C, H), a.dtype).at[r, r // C].set(a.reshape(-1))


def _edge_pass(hfull, sdp, z72):
    h = hfull[:, :128]
    asn = hfull[:, 128:136]
    adn = hfull[:, 136:144]
    h2e = jnp.concatenate([
        jnp.concatenate([h[:, :64], asn[:, :4], adn[:, :4]], axis=1),
        jnp.concatenate([h[:, 64:], asn[:, 4:], adn[:, 4:]], axis=1),
    ], axis=0)
    att2 = jnp.concatenate([
        jnp.concatenate([asn[:, :4], adn[:, :4]], axis=1),
        jnp.concatenate([asn[:, 4:], adn[:, 4:]], axis=1),
    ], axis=0)
    acc = _edge(h2e, att2, sdp, z72)
    out0, out1 = acc[:_N], acc[_NP:_NP + _N]
    n0 = out0[:, :64]
    n1 = out1[:, :64]
    dcat = jnp.concatenate([out0[:, 64:68], out1[:, 64:68]], axis=1)
    dexp = jnp.repeat(dcat, 16, axis=1)
    return n0, n1, dexp


def kernel(x, edge_index, W0, as0, ad0, b0, W1, as1, ad1, b1,
           W2, as2, ad2, b2, Wp, bp):
    loop = jnp.arange(_N, dtype=edge_index.dtype)
    pad = jnp.zeros((_EP - _EREAL,), edge_index.dtype)
    srcp = jnp.concatenate([edge_index[0], loop, pad]).astype(jnp.int32)
    dstp = jnp.concatenate([edge_index[1], loop, pad]).astype(jnp.int32)
    sdp = jnp.stack([srcp.reshape(16 * _NCH, _B),
                     dstp.reshape(16 * _NCH, _B)], axis=1).reshape(-1, _B)
    z72 = jnp.zeros((_NP, 72), jnp.float32)

    wf0 = jnp.concatenate(
        [W0, W0 @ _blockdiag(as0, 8, 16), W0 @ _blockdiag(ad0, 8, 16)], axis=1)
    wf1 = jnp.concatenate(
        [W1, W1 @ _blockdiag(as1, 8, 16), W1 @ _blockdiag(ad1, 8, 16)], axis=1)
    a2s = jnp.tile(as2.reshape(128, 1), (1, 8))
    a2d = jnp.tile(ad2.reshape(128, 1), (1, 8))
    wf2 = jnp.concatenate([W2, W2 @ a2s, W2 @ a2d], axis=1)

    hfull = _mm(x, wf0)
    n0, n1, dexp = _edge_pass(hfull, sdp, z72)
    hfull = _mm_fused(n0, n1, dexp, b0.reshape(1, 128), wf1)
    n0, n1, dexp = _edge_pass(hfull, sdp, z72)
    hfull = _mm_fused(n0, n1, dexp, b1.reshape(1, 128), wf2)
    n0, n1, dexp = _edge_pass(hfull, sdp, z72)
    return _mm_final(n0, n1, dexp, b2.reshape(1, 128),
                     Wp, bp.reshape(1, 128))
